# Initial kernel scaffold; baseline (speedup 1.0000x reference)
import jax
import jax.numpy as jnp
from jax.experimental import pallas as pl


def kernel(region_attributes, distance, edge_index, W0, al0, ar0, b0, W1, al1, ar1, b1, W2, al2, ar2, b2, resW2, lam1, lam2, lam3, beta, G):
    raise NotImplementedError("write your pallas kernel here")


# speedup vs baseline: 25.3586x; 25.3586x over previous
import functools

import jax
import jax.numpy as jnp
import numpy as np
from jax import lax
from jax.experimental import pallas as pl
from jax.experimental.pallas import tpu as pltpu
from jax.experimental.pallas import tpu_sc as plsc

_N = 2048
_E = 32768
_K = 128
_NC, _NS = 2, 16
_BN = 256
_BG = 256

_f32 = jnp.float32


def _prep0_body(x_ref, w_ref, alf_ref, arf_ref, s64_ref, aug_ref, el_ref,
                er_ref):
    feat = jnp.dot(x_ref[...], w_ref[...], preferred_element_type=_f32)
    el_ref[...] = jnp.dot(feat * alf_ref[...], s64_ref[...],
                          preferred_element_type=_f32,
                          precision=lax.Precision.HIGHEST)
    er_ref[...] = jnp.dot(feat * arf_ref[...], s64_ref[...],
                          preferred_element_type=_f32,
                          precision=lax.Precision.HIGHEST)
    aug_ref[:, :256] = feat
    aug_ref[:, 256:272] = jnp.ones((feat.shape[0], 16), _f32)


def _elu(x):
    return jnp.where(x > 0, x, jnp.exp(x) - 1.0)


def _comb_prep1_body(part_ref, bflat_ref, s16_ref, w_ref, alf_ref, arf_ref,
                     s64_ref, aug_ref, el_ref, er_ref, h_ref):
    acc = part_ref[0] + part_ref[1]
    inv = 1.0 / (acc[:, 256:272] + 1e-9)
    mult = jnp.dot(inv, s16_ref[...], preferred_element_type=_f32,
                   precision=lax.Precision.HIGHEST)
    rst = acc[:, :256] * mult + bflat_ref[...]
    h = _elu(rst)
    h_ref[...] = h
    feat = jnp.dot(h, w_ref[...], preferred_element_type=_f32)
    el_ref[...] = jnp.dot(feat * alf_ref[...], s64_ref[...],
                          preferred_element_type=_f32,
                          precision=lax.Precision.HIGHEST)
    er_ref[...] = jnp.dot(feat * arf_ref[...], s64_ref[...],
                          preferred_element_type=_f32,
                          precision=lax.Precision.HIGHEST)
    aug_ref[:, :256] = feat
    aug_ref[:, 256:272] = jnp.ones((feat.shape[0], 16), _f32)


def _comb_prep2_body(part_ref, hres_ref, bflat_ref, s16_ref, w2_ref, alf2_ref,
                     arf2_ref, s5_ref, resw2_ref, e5_ref,
                     aug_ref, el_ref, er_ref, res_ref):
    acc = part_ref[0] + part_ref[1]
    inv = 1.0 / (acc[:, 256:272] + 1e-9)
    mult = jnp.dot(inv, s16_ref[...], preferred_element_type=_f32,
                   precision=lax.Precision.HIGHEST)
    rst = acc[:, :256] * mult + hres_ref[...] + bflat_ref[...]
    h2 = _elu(rst)
    f2 = jnp.dot(h2, w2_ref[...], preferred_element_type=_f32)
    aug_ref[...] = f2 + e5_ref[...]
    el_ref[...] = jnp.dot(f2 * alf2_ref[...], s5_ref[...],
                          preferred_element_type=_f32,
                          precision=lax.Precision.HIGHEST)
    er_ref[...] = jnp.dot(f2 * arf2_ref[...], s5_ref[...],
                          preferred_element_type=_f32,
                          precision=lax.Precision.HIGHEST)
    res_ref[...] = jnp.dot(h2, resw2_ref[...], preferred_element_type=_f32)


def _emb_body(part_ref, res_ref, b2_ref, p5_ref, scal_ref):
    acc = part_ref[0] + part_ref[1]
    esum = jnp.dot(acc, p5_ref[...], preferred_element_type=_f32,
                  precision=lax.Precision.HIGHEST)
    emb = acc / (esum + 1e-9) + res_ref[...] + b2_ref[...]
    lane = lax.broadcasted_iota(jnp.int32, emb.shape, 1)
    e14 = jnp.where((lane >= 1) & (lane < 5), emb, 0.0)
    sq = jnp.sum(e14 * e14, axis=1, keepdims=True)
    la = jnp.log(jnp.abs(emb[:, 0:1]))
    scal_ref[...] = jnp.where(
        lane == 0, la, jnp.where(lane < 5, emb, jnp.where(lane == 5, sq, 0.0)))


def _grav_body(scal_ref, scalt_ref, dist_ref, prm_ref, out_ref):
    lam1 = prm_ref[0:1, 0:1]
    lam2 = prm_ref[0:1, 1:2]
    lam3 = prm_ref[0:1, 2:3]
    beta = prm_ref[0:1, 3:4]
    g = prm_ref[0:1, 4:5]
    la_i = scal_ref[:, 0:1]
    la_j = scalt_ref[0:1, :]
    acc = jnp.zeros((_BG, _BG), _f32)
    for kk in range(1, 5):
        d = scal_ref[:, kk:kk + 1] - scalt_ref[kk:kk + 1, :]
        acc = acc + d * d
    r2 = acc * 0.25 + beta * dist_ref[...]
    logr = 0.5 * jnp.log(jnp.maximum(r2, 1e-7))
    od = g * jnp.exp(lam1 * la_j + lam2 * la_i - lam3 * logr)
    gi = pl.program_id(0) * _BG + lax.broadcasted_iota(jnp.int32, (_BG, _BG), 0)
    gj = pl.program_id(1) * _BG + lax.broadcasted_iota(jnp.int32, (_BG, _BG), 1)
    out_ref[...] = jnp.where(gi == gj, 0.0, od)


def _make_sc_edge_kernel(h_heads, wcols):
    nv = wcols // 16
    dh = (wcols - 16) // h_heads if h_heads > 1 else 16
    epw = _E // (_NC * _NS)
    nch = epw // _K
    rows_pt = _N // _NS

    mesh = plsc.VectorSubcoreMesh(core_axis_name="c", subcore_axis_name="s")

    @functools.partial(
        pl.kernel,
        mesh=mesh,
        compiler_params=pltpu.CompilerParams(use_tc_tiling_on_sc=False),
        out_type=jax.ShapeDtypeStruct((_NC, _N, wcols), _f32),
        scratch_types=[
            pltpu.VMEM((nch, _K), jnp.int32),
            pltpu.VMEM((nch, _K), jnp.int32),
            pltpu.VMEM((_K, 16), _f32),
            pltpu.VMEM((_K, 16), _f32),
            pltpu.VMEM((_K * 16,), _f32),
            pltpu.VMEM((_K, wcols), _f32),
            pltpu.VMEM((_K, wcols), _f32),
            pltpu.VMEM_SHARED((_N, wcols), _f32),
            pltpu.SemaphoreType.DMA,
        ],
    )
    def sc_edge(src_hbm, dst_hbm, el_hbm, er_hbm, aug_hbm, zeros_hbm, out_hbm,
                src_v, dst_v, el_v, er_v, ee_v, rows_v, sc_v, acc_sh, sem):
        c = lax.axis_index("c")
        s = lax.axis_index("s")
        pltpu.sync_copy(zeros_hbm.at[pl.ds(s * rows_pt, rows_pt)], rows_v)
        pltpu.sync_copy(rows_v, acc_sh.at[pl.ds(s * rows_pt, rows_pt)])
        plsc.subcore_barrier()

        w = s * _NC + c
        pltpu.sync_copy(src_hbm.at[pl.ds(w * nch, nch)], src_v)
        pltpu.sync_copy(dst_hbm.at[pl.ds(w * nch, nch)], dst_v)
        iota16 = lax.iota(jnp.int32, 16)
        patt = lax.rem(iota16, h_heads)

        for j in range(nch):
            cp_el = pltpu.async_copy(el_hbm.at[src_v.at[j]], el_v, sem)
            cp_er = pltpu.async_copy(er_hbm.at[dst_v.at[j]], er_v, sem)
            cp_rows = pltpu.async_copy(aug_hbm.at[src_v.at[j]], rows_v, sem)
            cp_el.wait()
            cp_er.wait()

            def ee_body(k, carry):
                e = el_v[k] + er_v[k]
                e = jnp.where(e >= 0.0, e, e * 0.2)
                ee_v[pl.ds(k * 16, 16)] = jnp.exp(e)
                return carry

            lax.fori_loop(0, _K, ee_body, 0)
            cp_rows.wait()

            def scale_body(k, carry):
                base = k * 16
                eerow = ee_v[pl.ds(base, 16)]
                ms = [jnp.full((16,), eerow[hh], _f32)
                      for hh in range(h_heads)]
                mp = ms[0]
                for hh in range(1, h_heads):
                    mp = jnp.where(patt == hh, ms[hh], mp)
                for v in range(nv):
                    m = mp if v == nv - 1 else ms[(v * 16) // dh]
                    sc_v[k, pl.ds(v * 16, 16)] = (
                        rows_v[k, pl.ds(v * 16, 16)] * m)
                return carry

            lax.fori_loop(0, _K, scale_body, 0)
            pltpu.sync_copy(sc_v, acc_sh.at[dst_v.at[j]], add=True)

        plsc.subcore_barrier()
        pltpu.sync_copy(acc_sh.at[pl.ds(s * rows_pt, rows_pt)], rows_v)
        pltpu.sync_copy(rows_v, out_hbm.at[c, pl.ds(s * rows_pt, rows_pt)])

    return sc_edge


_sc_edge_272 = _make_sc_edge_kernel(4, 272)
_sc_edge_16 = _make_sc_edge_kernel(1, 16)


_S16 = np.equal(np.arange(256)[None, :] // 64,
                np.arange(16)[:, None]).astype(np.float32)
_S5 = np.zeros((16, 16), np.float32)
_S5[:5, 0] = 1.0
_P5 = np.zeros((16, 16), np.float32)
_P5[5, :] = 1.0
_E5 = np.zeros((1, 16), np.float32)
_E5[0, 5] = 1.0


def _prep0(x, w0, alf0, arf0, s64):
    return pl.pallas_call(
        _prep0_body,
        grid=(_N // _BN,),
        in_specs=[
            pl.BlockSpec((_BN, 128), lambda i: (i, 0)),
            pl.BlockSpec((128, 256), lambda i: (0, 0)),
            pl.BlockSpec((1, 256), lambda i: (0, 0)),
            pl.BlockSpec((1, 256), lambda i: (0, 0)),
            pl.BlockSpec((256, 16), lambda i: (0, 0)),
        ],
        out_specs=[
            pl.BlockSpec((_BN, 272), lambda i: (i, 0)),
            pl.BlockSpec((_BN, 16), lambda i: (i, 0)),
            pl.BlockSpec((_BN, 16), lambda i: (i, 0)),
        ],
        out_shape=[
            jax.ShapeDtypeStruct((_N, 272), _f32),
            jax.ShapeDtypeStruct((_N, 16), _f32),
            jax.ShapeDtypeStruct((_N, 16), _f32),
        ],
    )(x, w0, alf0, arf0, s64)


def _comb_prep1(part, bflat, s16, w1, alf1, arf1, s64):
    return pl.pallas_call(
        _comb_prep1_body,
        grid=(_N // _BN,),
        in_specs=[
            pl.BlockSpec((2, _BN, 272), lambda i: (0, i, 0)),
            pl.BlockSpec((1, 256), lambda i: (0, 0)),
            pl.BlockSpec((16, 256), lambda i: (0, 0)),
            pl.BlockSpec((256, 256), lambda i: (0, 0)),
            pl.BlockSpec((1, 256), lambda i: (0, 0)),
            pl.BlockSpec((1, 256), lambda i: (0, 0)),
            pl.BlockSpec((256, 16), lambda i: (0, 0)),
        ],
        out_specs=[
            pl.BlockSpec((_BN, 272), lambda i: (i, 0)),
            pl.BlockSpec((_BN, 16), lambda i: (i, 0)),
            pl.BlockSpec((_BN, 16), lambda i: (i, 0)),
            pl.BlockSpec((_BN, 256), lambda i: (i, 0)),
        ],
        out_shape=[
            jax.ShapeDtypeStruct((_N, 272), _f32),
            jax.ShapeDtypeStruct((_N, 16), _f32),
            jax.ShapeDtypeStruct((_N, 16), _f32),
            jax.ShapeDtypeStruct((_N, 256), _f32),
        ],
    )(part, bflat, s16, w1, alf1, arf1, s64)


def _comb_prep2(part, h1, bflat, s16, w2p, alf2, arf2, s5, resw2p, e5):
    return pl.pallas_call(
        _comb_prep2_body,
        grid=(_N // _BN,),
        in_specs=[
            pl.BlockSpec((2, _BN, 272), lambda i: (0, i, 0)),
            pl.BlockSpec((_BN, 256), lambda i: (i, 0)),
            pl.BlockSpec((1, 256), lambda i: (0, 0)),
            pl.BlockSpec((16, 256), lambda i: (0, 0)),
            pl.BlockSpec((256, 16), lambda i: (0, 0)),
            pl.BlockSpec((1, 16), lambda i: (0, 0)),
            pl.BlockSpec((1, 16), lambda i: (0, 0)),
            pl.BlockSpec((16, 16), lambda i: (0, 0)),
            pl.BlockSpec((256, 16), lambda i: (0, 0)),
            pl.BlockSpec((1, 16), lambda i: (0, 0)),
        ],
        out_specs=[
            pl.BlockSpec((_BN, 16), lambda i: (i, 0)),
            pl.BlockSpec((_BN, 16), lambda i: (i, 0)),
            pl.BlockSpec((_BN, 16), lambda i: (i, 0)),
            pl.BlockSpec((_BN, 16), lambda i: (i, 0)),
        ],
        out_shape=[
            jax.ShapeDtypeStruct((_N, 16), _f32),
            jax.ShapeDtypeStruct((_N, 16), _f32),
            jax.ShapeDtypeStruct((_N, 16), _f32),
            jax.ShapeDtypeStruct((_N, 16), _f32),
        ],
    )(part, h1, bflat, s16, w2p, alf2, arf2, s5, resw2p, e5)


def _emb_scal(part2, res2, b2p, p5):
    return pl.pallas_call(
        _emb_body,
        grid=(_N // _BN,),
        in_specs=[
            pl.BlockSpec((2, _BN, 16), lambda i: (0, i, 0)),
            pl.BlockSpec((_BN, 16), lambda i: (i, 0)),
            pl.BlockSpec((1, 16), lambda i: (0, 0)),
            pl.BlockSpec((16, 16), lambda i: (0, 0)),
        ],
        out_specs=pl.BlockSpec((_BN, 16), lambda i: (i, 0)),
        out_shape=jax.ShapeDtypeStruct((_N, 16), _f32),
    )(part2, res2, b2p, p5)


def _gravity(scal, scalt, distance, prm):
    return pl.pallas_call(
        _grav_body,
        grid=(_N // _BG, _N // _BG),
        in_specs=[
            pl.BlockSpec((_BG, 16), lambda i, j: (i, 0)),
            pl.BlockSpec((16, _BG), lambda i, j: (0, j)),
            pl.BlockSpec((_BG, _BG), lambda i, j: (i, j)),
            pl.BlockSpec((1, 8), lambda i, j: (0, 0)),
        ],
        out_specs=pl.BlockSpec((_BG, _BG), lambda i, j: (i, j)),
        out_shape=jax.ShapeDtypeStruct((_N, _N), _f32),
    )(scal, scalt, distance, prm)


def kernel(region_attributes, distance, edge_index, W0, al0, ar0, b0,
           W1, al1, ar1, b1, W2, al2, ar2, b2, resW2,
           lam1, lam2, lam3, beta, G):
    x = region_attributes.astype(_f32)
    src2d = edge_index[0].astype(jnp.int32).reshape(_E // _K, _K)
    dst2d = edge_index[1].astype(jnp.int32).reshape(_E // _K, _K)

    alf0, arf0 = al0.reshape(1, 256), ar0.reshape(1, 256)
    alf1, arf1 = al1.reshape(1, 256), ar1.reshape(1, 256)
    alf2 = jnp.zeros((1, 16), _f32).at[0, :5].set(al2[0])
    arf2 = jnp.zeros((1, 16), _f32).at[0, :5].set(ar2[0])
    s64 = jnp.asarray(_S16).T
    s5 = jnp.asarray(_S5)
    w2p = jnp.zeros((256, 16), _f32).at[:, :5].set(W2)
    resw2p = jnp.zeros((256, 16), _f32).at[:, :5].set(resW2)
    b0flat = b0.reshape(1, 256).astype(_f32)
    b1flat = b1.reshape(1, 256).astype(_f32)
    b2p = jnp.zeros((1, 16), _f32).at[0, :5].set(b2[0])
    s16 = jnp.asarray(_S16)
    p5 = jnp.asarray(_P5)
    e5 = jnp.asarray(_E5)
    prm = jnp.concatenate(
        [lam1, lam2, lam3, beta, G, jnp.zeros((3,), _f32)]).reshape(1, 8)
    zeros272 = jnp.zeros((_N, 272), _f32)
    zeros16 = jnp.zeros((_N, 16), _f32)

    _edge272, _edge16 = _sc_edge_272, _sc_edge_16

    aug0, el0, er0 = _prep0(x, W0, alf0, arf0, s64)
    part0 = _edge272(src2d, dst2d, el0, er0, aug0, zeros272)
    aug1, el1, er1, h1 = _comb_prep1(part0, b0flat, s16, W1, alf1, arf1, s64)
    part1 = _edge272(src2d, dst2d, el1, er1, aug1, zeros272)
    aug2, el2, er2, res2 = _comb_prep2(part1, h1, b1flat, s16,
                                       w2p, alf2, arf2, s5, resw2p, e5)
    part2 = _edge16(src2d, dst2d, el2, er2, aug2, zeros16)
    scal = _emb_scal(part2, res2, b2p, p5)
    scalt = scal.T
    return _gravity(scal, scalt, distance, prm)


# speedup vs baseline: 28.5467x; 1.1257x over previous
import functools

import jax
import jax.numpy as jnp
import numpy as np
from jax import lax
from jax.experimental import pallas as pl
from jax.experimental.pallas import tpu as pltpu
from jax.experimental.pallas import tpu_sc as plsc

_N = 2048
_E = 32768
_K = 64
_NC, _NS = 2, 16
_BN = 256
_BG = 256

_f32 = jnp.float32


def _prep0_body(x_ref, w_ref, alf_ref, arf_ref, s64_ref, aug_ref, el_ref,
                er_ref):
    feat = jnp.dot(x_ref[...], w_ref[...], preferred_element_type=_f32)
    el_ref[...] = jnp.dot(feat * alf_ref[...], s64_ref[...],
                          preferred_element_type=_f32,
                          precision=lax.Precision.HIGHEST)
    er_ref[...] = jnp.dot(feat * arf_ref[...], s64_ref[...],
                          preferred_element_type=_f32,
                          precision=lax.Precision.HIGHEST)
    aug_ref[:, :256] = feat
    aug_ref[:, 256:272] = jnp.ones((feat.shape[0], 16), _f32)


def _elu(x):
    return jnp.where(x > 0, x, jnp.exp(x) - 1.0)


def _comb_prep1_body(part_ref, bflat_ref, s16_ref, w_ref, alf_ref, arf_ref,
                     s64_ref, aug_ref, el_ref, er_ref, h_ref):
    acc = part_ref[0] + part_ref[1]
    inv = 1.0 / (acc[:, 256:272] + 1e-9)
    mult = jnp.dot(inv, s16_ref[...], preferred_element_type=_f32,
                   precision=lax.Precision.HIGHEST)
    rst = acc[:, :256] * mult + bflat_ref[...]
    h = _elu(rst)
    h_ref[...] = h
    feat = jnp.dot(h, w_ref[...], preferred_element_type=_f32)
    el_ref[...] = jnp.dot(feat * alf_ref[...], s64_ref[...],
                          preferred_element_type=_f32,
                          precision=lax.Precision.HIGHEST)
    er_ref[...] = jnp.dot(feat * arf_ref[...], s64_ref[...],
                          preferred_element_type=_f32,
                          precision=lax.Precision.HIGHEST)
    aug_ref[:, :256] = feat
    aug_ref[:, 256:272] = jnp.ones((feat.shape[0], 16), _f32)


def _comb_prep2_body(part_ref, hres_ref, bflat_ref, s16_ref, w2_ref, alf2_ref,
                     arf2_ref, s5_ref, resw2_ref, e5_ref,
                     aug_ref, el_ref, er_ref, res_ref):
    acc = part_ref[0] + part_ref[1]
    inv = 1.0 / (acc[:, 256:272] + 1e-9)
    mult = jnp.dot(inv, s16_ref[...], preferred_element_type=_f32,
                   precision=lax.Precision.HIGHEST)
    rst = acc[:, :256] * mult + hres_ref[...] + bflat_ref[...]
    h2 = _elu(rst)
    f2 = jnp.dot(h2, w2_ref[...], preferred_element_type=_f32)
    aug_ref[...] = f2 + e5_ref[...]
    el_ref[...] = jnp.dot(f2 * alf2_ref[...], s5_ref[...],
                          preferred_element_type=_f32,
                          precision=lax.Precision.HIGHEST)
    er_ref[...] = jnp.dot(f2 * arf2_ref[...], s5_ref[...],
                          preferred_element_type=_f32,
                          precision=lax.Precision.HIGHEST)
    res_ref[...] = jnp.dot(h2, resw2_ref[...], preferred_element_type=_f32)


def _emb_body(part_ref, res_ref, b2_ref, p5_ref, scal_ref):
    acc = part_ref[0] + part_ref[1]
    esum = jnp.dot(acc, p5_ref[...], preferred_element_type=_f32,
                  precision=lax.Precision.HIGHEST)
    emb = acc / (esum + 1e-9) + res_ref[...] + b2_ref[...]
    lane = lax.broadcasted_iota(jnp.int32, emb.shape, 1)
    e14 = jnp.where((lane >= 1) & (lane < 5), emb, 0.0)
    sq = jnp.sum(e14 * e14, axis=1, keepdims=True)
    la = jnp.log(jnp.abs(emb[:, 0:1]))
    scal_ref[...] = jnp.where(
        lane == 0, la, jnp.where(lane < 5, emb, jnp.where(lane == 5, sq, 0.0)))


def _grav_body(scal_ref, scalt_ref, dist_ref, prm_ref, out_ref):
    lam1 = prm_ref[0:1, 0:1]
    lam2 = prm_ref[0:1, 1:2]
    lam3 = prm_ref[0:1, 2:3]
    beta = prm_ref[0:1, 3:4]
    g = prm_ref[0:1, 4:5]
    la_i = scal_ref[:, 0:1]
    la_j = scalt_ref[0:1, :]
    acc = jnp.zeros((_BG, _BG), _f32)
    for kk in range(1, 5):
        d = scal_ref[:, kk:kk + 1] - scalt_ref[kk:kk + 1, :]
        acc = acc + d * d
    r2 = acc * 0.25 + beta * dist_ref[...]
    logr = 0.5 * jnp.log(jnp.maximum(r2, 1e-7))
    od = g * jnp.exp(lam1 * la_j + lam2 * la_i - lam3 * logr)
    gi = pl.program_id(0) * _BG + lax.broadcasted_iota(jnp.int32, (_BG, _BG), 0)
    gj = pl.program_id(1) * _BG + lax.broadcasted_iota(jnp.int32, (_BG, _BG), 1)
    out_ref[...] = jnp.where(gi == gj, 0.0, od)


def _make_sc_edge_kernel(h_heads, wcols, kc=_K):
    nv = wcols // 16
    dh = (wcols - 16) // h_heads if h_heads > 1 else 16
    epw = _E // (_NC * _NS)
    nch = epw // kc
    rows_pt = _N // _NS

    mesh = plsc.VectorSubcoreMesh(core_axis_name="c", subcore_axis_name="s")

    @functools.partial(
        pl.kernel,
        mesh=mesh,
        compiler_params=pltpu.CompilerParams(use_tc_tiling_on_sc=False),
        out_type=jax.ShapeDtypeStruct((_NC, _N, wcols), _f32),
        scratch_types=[
            pltpu.VMEM((nch, kc), jnp.int32),
            pltpu.VMEM((nch, kc), jnp.int32),
            pltpu.VMEM((2, kc, 16), _f32),
            pltpu.VMEM((2, kc, 16), _f32),
            pltpu.VMEM((kc * 16,), _f32),
            pltpu.VMEM((2, kc, wcols), _f32),
            pltpu.VMEM((2, kc, wcols), _f32),
            pltpu.VMEM_SHARED((_N, wcols), _f32),
            pltpu.SemaphoreType.DMA,
            pltpu.SemaphoreType.DMA,
            pltpu.SemaphoreType.DMA,
            pltpu.SemaphoreType.DMA,
        ],
    )
    def sc_edge(src_hbm, dst_hbm, el_hbm, er_hbm, aug_hbm, zeros_hbm, out_hbm,
                src_v, dst_v, el_v, er_v, ee_v, rows_v, sc_v, acc_sh,
                gsem0, gsem1, ssem0, ssem1):
        c = lax.axis_index("c")
        s = lax.axis_index("s")
        gsems = (gsem0, gsem1)
        ssems = (ssem0, ssem1)
        half = rows_pt // 2
        for q in range(2):
            pltpu.sync_copy(
                zeros_hbm.at[pl.ds(s * rows_pt + q * half, half)],
                rows_v.at[0])
            pltpu.sync_copy(
                rows_v.at[0],
                acc_sh.at[pl.ds(s * rows_pt + q * half, half)])
        plsc.subcore_barrier()

        w = s * _NC + c
        pltpu.sync_copy(src_hbm.at[pl.ds(w * nch, nch)], src_v)
        pltpu.sync_copy(dst_hbm.at[pl.ds(w * nch, nch)], dst_v)
        iota16 = lax.iota(jnp.int32, 16)
        patt = lax.rem(iota16, h_heads)

        def issue_gathers(j, b):
            return (
                pltpu.async_copy(el_hbm.at[src_v.at[j]], el_v.at[b], gsems[b]),
                pltpu.async_copy(er_hbm.at[dst_v.at[j]], er_v.at[b], gsems[b]),
                pltpu.async_copy(aug_hbm.at[src_v.at[j]], rows_v.at[b],
                                 gsems[b]),
            )

        pend_g = issue_gathers(0, 0)
        pend_s = [None, None]
        for j in range(nch):
            b = j % 2
            for cp in pend_g:
                cp.wait()
            if j + 1 < nch:
                pend_g = issue_gathers(j + 1, 1 - b)

            def ee_body(k, carry):
                e = el_v[b, k] + er_v[b, k]
                e = jnp.where(e >= 0.0, e, e * 0.2)
                ee_v[pl.ds(k * 16, 16)] = jnp.exp(e)
                return carry

            lax.fori_loop(0, kc, ee_body, 0)

            if pend_s[b] is not None:
                pend_s[b].wait()

            def scale_body(k, carry):
                base = k * 16
                eerow = ee_v[pl.ds(base, 16)]
                ms = [jnp.full((16,), eerow[hh], _f32)
                      for hh in range(h_heads)]
                mp = ms[0]
                for hh in range(1, h_heads):
                    mp = jnp.where(patt == hh, ms[hh], mp)
                for v in range(nv):
                    m = mp if v == nv - 1 else ms[(v * 16) // dh]
                    sc_v[b, k, pl.ds(v * 16, 16)] = (
                        rows_v[b, k, pl.ds(v * 16, 16)] * m)
                return carry

            lax.fori_loop(0, kc, scale_body, 0)
            pend_s[b] = pltpu.async_copy(
                sc_v.at[b], acc_sh.at[dst_v.at[j]], ssems[b], add=True)

        for cp in pend_s:
            if cp is not None:
                cp.wait()
        plsc.subcore_barrier()
        for q in range(2):
            pltpu.sync_copy(
                acc_sh.at[pl.ds(s * rows_pt + q * half, half)],
                rows_v.at[0])
            pltpu.sync_copy(
                rows_v.at[0],
                out_hbm.at[c, pl.ds(s * rows_pt + q * half, half)])

    return sc_edge


_sc_edge_272 = _make_sc_edge_kernel(4, 272)
_sc_edge_16 = _make_sc_edge_kernel(1, 16)


_S16 = np.equal(np.arange(256)[None, :] // 64,
                np.arange(16)[:, None]).astype(np.float32)
_S5 = np.zeros((16, 16), np.float32)
_S5[:5, 0] = 1.0
_P5 = np.zeros((16, 16), np.float32)
_P5[5, :] = 1.0
_E5 = np.zeros((1, 16), np.float32)
_E5[0, 5] = 1.0


def _prep0(x, w0, alf0, arf0, s64):
    return pl.pallas_call(
        _prep0_body,
        grid=(_N // _BN,),
        in_specs=[
            pl.BlockSpec((_BN, 128), lambda i: (i, 0)),
            pl.BlockSpec((128, 256), lambda i: (0, 0)),
            pl.BlockSpec((1, 256), lambda i: (0, 0)),
            pl.BlockSpec((1, 256), lambda i: (0, 0)),
            pl.BlockSpec((256, 16), lambda i: (0, 0)),
        ],
        out_specs=[
            pl.BlockSpec((_BN, 272), lambda i: (i, 0)),
            pl.BlockSpec((_BN, 16), lambda i: (i, 0)),
            pl.BlockSpec((_BN, 16), lambda i: (i, 0)),
        ],
        out_shape=[
            jax.ShapeDtypeStruct((_N, 272), _f32),
            jax.ShapeDtypeStruct((_N, 16), _f32),
            jax.ShapeDtypeStruct((_N, 16), _f32),
        ],
    )(x, w0, alf0, arf0, s64)


def _comb_prep1(part, bflat, s16, w1, alf1, arf1, s64):
    return pl.pallas_call(
        _comb_prep1_body,
        grid=(_N // _BN,),
        in_specs=[
            pl.BlockSpec((2, _BN, 272), lambda i: (0, i, 0)),
            pl.BlockSpec((1, 256), lambda i: (0, 0)),
            pl.BlockSpec((16, 256), lambda i: (0, 0)),
            pl.BlockSpec((256, 256), lambda i: (0, 0)),
            pl.BlockSpec((1, 256), lambda i: (0, 0)),
            pl.BlockSpec((1, 256), lambda i: (0, 0)),
            pl.BlockSpec((256, 16), lambda i: (0, 0)),
        ],
        out_specs=[
            pl.BlockSpec((_BN, 272), lambda i: (i, 0)),
            pl.BlockSpec((_BN, 16), lambda i: (i, 0)),
            pl.BlockSpec((_BN, 16), lambda i: (i, 0)),
            pl.BlockSpec((_BN, 256), lambda i: (i, 0)),
        ],
        out_shape=[
            jax.ShapeDtypeStruct((_N, 272), _f32),
            jax.ShapeDtypeStruct((_N, 16), _f32),
            jax.ShapeDtypeStruct((_N, 16), _f32),
            jax.ShapeDtypeStruct((_N, 256), _f32),
        ],
    )(part, bflat, s16, w1, alf1, arf1, s64)


def _comb_prep2(part, h1, bflat, s16, w2p, alf2, arf2, s5, resw2p, e5):
    return pl.pallas_call(
        _comb_prep2_body,
        grid=(_N // _BN,),
        in_specs=[
            pl.BlockSpec((2, _BN, 272), lambda i: (0, i, 0)),
            pl.BlockSpec((_BN, 256), lambda i: (i, 0)),
            pl.BlockSpec((1, 256), lambda i: (0, 0)),
            pl.BlockSpec((16, 256), lambda i: (0, 0)),
            pl.BlockSpec((256, 16), lambda i: (0, 0)),
            pl.BlockSpec((1, 16), lambda i: (0, 0)),
            pl.BlockSpec((1, 16), lambda i: (0, 0)),
            pl.BlockSpec((16, 16), lambda i: (0, 0)),
            pl.BlockSpec((256, 16), lambda i: (0, 0)),
            pl.BlockSpec((1, 16), lambda i: (0, 0)),
        ],
        out_specs=[
            pl.BlockSpec((_BN, 16), lambda i: (i, 0)),
            pl.BlockSpec((_BN, 16), lambda i: (i, 0)),
            pl.BlockSpec((_BN, 16), lambda i: (i, 0)),
            pl.BlockSpec((_BN, 16), lambda i: (i, 0)),
        ],
        out_shape=[
            jax.ShapeDtypeStruct((_N, 16), _f32),
            jax.ShapeDtypeStruct((_N, 16), _f32),
            jax.ShapeDtypeStruct((_N, 16), _f32),
            jax.ShapeDtypeStruct((_N, 16), _f32),
        ],
    )(part, h1, bflat, s16, w2p, alf2, arf2, s5, resw2p, e5)


def _emb_scal(part2, res2, b2p, p5):
    return pl.pallas_call(
        _emb_body,
        grid=(_N // _BN,),
        in_specs=[
            pl.BlockSpec((2, _BN, 16), lambda i: (0, i, 0)),
            pl.BlockSpec((_BN, 16), lambda i: (i, 0)),
            pl.BlockSpec((1, 16), lambda i: (0, 0)),
            pl.BlockSpec((16, 16), lambda i: (0, 0)),
        ],
        out_specs=pl.BlockSpec((_BN, 16), lambda i: (i, 0)),
        out_shape=jax.ShapeDtypeStruct((_N, 16), _f32),
    )(part2, res2, b2p, p5)


def _gravity(scal, scalt, distance, prm):
    return pl.pallas_call(
        _grav_body,
        grid=(_N // _BG, _N // _BG),
        in_specs=[
            pl.BlockSpec((_BG, 16), lambda i, j: (i, 0)),
            pl.BlockSpec((16, _BG), lambda i, j: (0, j)),
            pl.BlockSpec((_BG, _BG), lambda i, j: (i, j)),
            pl.BlockSpec((1, 8), lambda i, j: (0, 0)),
        ],
        out_specs=pl.BlockSpec((_BG, _BG), lambda i, j: (i, j)),
        out_shape=jax.ShapeDtypeStruct((_N, _N), _f32),
    )(scal, scalt, distance, prm)


def kernel(region_attributes, distance, edge_index, W0, al0, ar0, b0,
           W1, al1, ar1, b1, W2, al2, ar2, b2, resW2,
           lam1, lam2, lam3, beta, G):
    x = region_attributes.astype(_f32)
    src2d = edge_index[0].astype(jnp.int32).reshape(_E // _K, _K)
    dst2d = edge_index[1].astype(jnp.int32).reshape(_E // _K, _K)

    alf0, arf0 = al0.reshape(1, 256), ar0.reshape(1, 256)
    alf1, arf1 = al1.reshape(1, 256), ar1.reshape(1, 256)
    alf2 = jnp.zeros((1, 16), _f32).at[0, :5].set(al2[0])
    arf2 = jnp.zeros((1, 16), _f32).at[0, :5].set(ar2[0])
    s64 = jnp.asarray(_S16).T
    s5 = jnp.asarray(_S5)
    w2p = jnp.zeros((256, 16), _f32).at[:, :5].set(W2)
    resw2p = jnp.zeros((256, 16), _f32).at[:, :5].set(resW2)
    b0flat = b0.reshape(1, 256).astype(_f32)
    b1flat = b1.reshape(1, 256).astype(_f32)
    b2p = jnp.zeros((1, 16), _f32).at[0, :5].set(b2[0])
    s16 = jnp.asarray(_S16)
    p5 = jnp.asarray(_P5)
    e5 = jnp.asarray(_E5)
    prm = jnp.concatenate(
        [lam1, lam2, lam3, beta, G, jnp.zeros((3,), _f32)]).reshape(1, 8)
    zeros272 = jnp.zeros((_N, 272), _f32)
    zeros16 = jnp.zeros((_N, 16), _f32)

    _edge272, _edge16 = _sc_edge_272, _sc_edge_16

    aug0, el0, er0 = _prep0(x, W0, alf0, arf0, s64)
    part0 = _edge272(src2d, dst2d, el0, er0, aug0, zeros272)
    aug1, el1, er1, h1 = _comb_prep1(part0, b0flat, s16, W1, alf1, arf1, s64)
    part1 = _edge272(src2d, dst2d, el1, er1, aug1, zeros272)
    aug2, el2, er2, res2 = _comb_prep2(part1, h1, b1flat, s16,
                                       w2p, alf2, arf2, s5, resw2p, e5)
    part2 = _edge16(src2d, dst2d, el2, er2, aug2, zeros16)
    scal = _emb_scal(part2, res2, b2p, p5)
    scalt = scal.T
    return _gravity(scal, scalt, distance, prm)
